# baseline (speedup 1.0000x reference)
import functools

import jax
import jax.numpy as jnp
from jax import lax
from jax.experimental import pallas as pl
from jax.experimental.pallas import tpu as pltpu
from jax.experimental.pallas import tpu_sc as plsc

_CHUNK = 128


@functools.lru_cache(maxsize=None)
def _make_gather(n_half, d):
    info = plsc.get_sparse_core_info()
    nc, ns = info.num_cores, info.num_subcores
    nw = nc * ns
    assert n_half % (nw * _CHUNK) == 0
    rows_per_w = n_half // nw
    chunks_per_w = rows_per_w // _CHUNK
    idx_rows = n_half // _CHUNK
    idx_rows_per_w = idx_rows // nw
    mesh = plsc.VectorSubcoreMesh(core_axis_name="c", subcore_axis_name="s")

    @functools.partial(
        pl.kernel,
        mesh=mesh,
        out_type=jax.ShapeDtypeStruct((n_half, 2 * d), jnp.float32),
        compiler_params=pltpu.CompilerParams(use_tc_tiling_on_sc=False),
        scratch_types=[
            pltpu.VMEM((2 * idx_rows_per_w, _CHUNK), jnp.int32),
            pltpu.VMEM((2, _CHUNK, d), jnp.float32),
            pltpu.VMEM((2, _CHUNK, d), jnp.float32),
            pltpu.SemaphoreType.DMA,
            pltpu.SemaphoreType.DMA,
        ],
    )
    def gather_kernel(table_hbm, idx2d_hbm, out_hbm, idx_v, gl, gr,
                      sem_g, sem_out):
        wid = lax.axis_index("s") * nc + lax.axis_index("c")
        base = wid * rows_per_w

        pltpu.sync_copy(
            idx2d_hbm.at[pl.ds(wid * idx_rows_per_w, idx_rows_per_w)],
            idx_v.at[pl.ds(0, idx_rows_per_w)])
        pltpu.sync_copy(
            idx2d_hbm.at[pl.ds(idx_rows + wid * idx_rows_per_w,
                               idx_rows_per_w)],
            idx_v.at[pl.ds(idx_rows_per_w, idx_rows_per_w)])

        def chunk_start(c, buf):
            cl = pltpu.async_copy(
                table_hbm.at[idx_v.at[c]], gl.at[buf], sem_g)
            cr = pltpu.async_copy(
                table_hbm.at[idx_v.at[idx_rows_per_w + c]], gr.at[buf], sem_g)
            return cl, cr

        def store_start(c, buf):
            r0 = base + c * _CHUNK
            pltpu.async_copy(
                gl.at[buf], out_hbm.at[pl.ds(r0, _CHUNK), pl.ds(0, d)],
                sem_out)
            pltpu.async_copy(
                gr.at[buf], out_hbm.at[pl.ds(r0, _CHUNK), pl.ds(d, d)],
                sem_out)

        def store_wait(buf):
            for _ in range(2):
                pltpu.make_async_copy(
                    gl.at[buf],
                    out_hbm.at[pl.ds(base, _CHUNK), pl.ds(0, d)],
                    sem_out).wait()

        def run_chunk(c, buf):
            cl, cr = chunk_start(c, buf)
            cl.wait()
            cr.wait()
            store_start(c, buf)

        run_chunk(0, 0)
        run_chunk(1, 1)

        def body(g, carry):
            c0 = 2 * g
            store_wait(0)
            run_chunk(c0, 0)
            store_wait(1)
            run_chunk(c0 + 1, 1)
            return carry

        lax.fori_loop(1, (chunks_per_w - 1) // 2, body, 0)
        store_wait(0)
        run_chunk(chunks_per_w - 1, 0)
        store_wait(1)
        store_wait(0)

    return gather_kernel


@functools.lru_cache(maxsize=None)
def _make_retile(b, m, k, d):
    kh = k // 2
    bb = b // 128

    def body(in_ref, out_ref):
        x = in_ref[...][:, 0]
        ii = lax.broadcasted_iota(jnp.int32, (128, 128), 0)
        jj = lax.broadcasted_iota(jnp.int32, (128, 128), 1)
        ident = jnp.where(ii == jj, 1.0, 0.0).astype(jnp.float32)
        ys = [
            lax.dot_general(
                x[:, q, :], ident, (((0,), (0,)), ((), ())),
                preferred_element_type=jnp.float32).reshape(2, d, 128)
            for q in range(kh)
        ]
        out_ref[...] = jnp.stack(ys, axis=1)[None]

    def retile(mid4d):
        return pl.pallas_call(
            body,
            grid=(bb, m),
            in_specs=[pl.BlockSpec(
                (128, 1, kh, 2 * d), lambda ib, im: (ib, im, 0, 0))],
            out_specs=pl.BlockSpec(
                (1, 2, kh, d, 128), lambda ib, im: (im, 0, 0, 0, ib)),
            out_shape=jax.ShapeDtypeStruct((m, 2, kh, d, b), jnp.float32),
        )(mid4d)

    return retile


def kernel(entity_cand_eid, table):
    b, m, k = entity_cand_eid.shape
    d = table.shape[1]
    kh = k // 2
    idxl = entity_cand_eid[:, :, :kh].reshape(-1, _CHUNK)
    idxr = entity_cand_eid[:, :, kh:].reshape(-1, _CHUNK)
    idx2d = jnp.concatenate([idxl, idxr])
    n_half = b * m * kh
    mid = _make_gather(n_half, d)(table, idx2d)
    mid4d = mid.reshape(b, m, kh, 2 * d)
    out_t = _make_retile(b, m, k, d)(mid4d)
    out_t = out_t.reshape(m, k, d, b)
    return jnp.transpose(out_t, (3, 0, 1, 2))


# speedup vs baseline: 1.1115x; 1.1115x over previous
import functools

import jax
import jax.numpy as jnp
from jax import lax
from jax.experimental import pallas as pl
from jax.experimental.pallas import tpu as pltpu
from jax.experimental.pallas import tpu_sc as plsc

_CHUNK = 128


@functools.lru_cache(maxsize=None)
def _make_gather(n_half, d):
    info = plsc.get_sparse_core_info()
    nc, ns = info.num_cores, info.num_subcores
    nw = nc * ns
    assert n_half % (nw * _CHUNK) == 0
    rows_per_w = n_half // nw
    chunks_per_w = rows_per_w // _CHUNK
    idx_rows = n_half // _CHUNK
    idx_rows_per_w = idx_rows // nw
    mesh = plsc.VectorSubcoreMesh(core_axis_name="c", subcore_axis_name="s")

    @functools.partial(
        pl.kernel,
        mesh=mesh,
        out_type=jax.ShapeDtypeStruct((n_half, 2 * d), jnp.float32),
        compiler_params=pltpu.CompilerParams(use_tc_tiling_on_sc=False),
        scratch_types=[
            pltpu.VMEM((2 * idx_rows_per_w, _CHUNK), jnp.int32),
            pltpu.VMEM((2, _CHUNK, d), jnp.float32),
            pltpu.VMEM((2, _CHUNK, d), jnp.float32),
            pltpu.SemaphoreType.DMA,
            pltpu.SemaphoreType.DMA,
        ],
    )
    def gather_kernel(table_hbm, idx2d_hbm, out_hbm, idx_v, gl, gr,
                      sem_g, sem_out):
        wid = lax.axis_index("s") * nc + lax.axis_index("c")
        base = wid * rows_per_w

        pltpu.sync_copy(
            idx2d_hbm.at[pl.ds(wid * idx_rows_per_w, idx_rows_per_w)],
            idx_v.at[pl.ds(0, idx_rows_per_w)])
        pltpu.sync_copy(
            idx2d_hbm.at[pl.ds(idx_rows + wid * idx_rows_per_w,
                               idx_rows_per_w)],
            idx_v.at[pl.ds(idx_rows_per_w, idx_rows_per_w)])

        def chunk_start(c, buf):
            cl = pltpu.async_copy(
                table_hbm.at[idx_v.at[c]], gl.at[buf], sem_g)
            cr = pltpu.async_copy(
                table_hbm.at[idx_v.at[idx_rows_per_w + c]], gr.at[buf], sem_g)
            return cl, cr

        def store_start(c, buf):
            r0 = base + c * _CHUNK
            pltpu.async_copy(
                gl.at[buf], out_hbm.at[pl.ds(r0, _CHUNK), pl.ds(0, d)],
                sem_out)
            pltpu.async_copy(
                gr.at[buf], out_hbm.at[pl.ds(r0, _CHUNK), pl.ds(d, d)],
                sem_out)

        def store_wait(buf):
            for _ in range(2):
                pltpu.make_async_copy(
                    gl.at[buf],
                    out_hbm.at[pl.ds(base, _CHUNK), pl.ds(0, d)],
                    sem_out).wait()

        def run_chunk(c, buf):
            cl, cr = chunk_start(c, buf)
            cl.wait()
            cr.wait()
            store_start(c, buf)

        run_chunk(0, 0)
        run_chunk(1, 1)

        def body(g, carry):
            c0 = 2 * g
            store_wait(0)
            run_chunk(c0, 0)
            store_wait(1)
            run_chunk(c0 + 1, 1)
            return carry

        lax.fori_loop(1, (chunks_per_w - 1) // 2, body, 0)
        store_wait(0)
        run_chunk(chunks_per_w - 1, 0)
        store_wait(1)
        store_wait(0)

    return gather_kernel


@functools.lru_cache(maxsize=None)
def _make_retile(b, m, k, d):
    kh = k // 2
    bb = b // 128

    def body(in_ref, out_ref):
        x = in_ref[...][:, 0]
        for q in range(kh):
            for ib in range(bb):
                t = jnp.transpose(x[ib * 128:(ib + 1) * 128, q, :])
                out_ref[0, :, q, :, ib * 128:(ib + 1) * 128] = (
                    t.reshape(2, d, 128))

    def retile(mid4d):
        return pl.pallas_call(
            body,
            grid=(m,),
            in_specs=[pl.BlockSpec(
                (b, 1, kh, 2 * d), lambda im: (0, im, 0, 0))],
            out_specs=pl.BlockSpec(
                (1, 2, kh, d, b), lambda im: (im, 0, 0, 0, 0)),
            out_shape=jax.ShapeDtypeStruct((m, 2, kh, d, b), jnp.float32),
        )(mid4d)

    return retile


def kernel(entity_cand_eid, table):
    b, m, k = entity_cand_eid.shape
    d = table.shape[1]
    kh = k // 2
    idxl = entity_cand_eid[:, :, :kh].reshape(-1, _CHUNK)
    idxr = entity_cand_eid[:, :, kh:].reshape(-1, _CHUNK)
    idx2d = jnp.concatenate([idxl, idxr])
    n_half = b * m * kh
    mid = _make_gather(n_half, d)(table, idx2d)
    mid4d = mid.reshape(b, m, kh, 2 * d)
    out_t = _make_retile(b, m, k, d)(mid4d)
    out_t = out_t.reshape(m, k, d, b)
    return jnp.transpose(out_t, (3, 0, 1, 2))
